# baseline (speedup 1.0000x reference)
import functools

import jax
import jax.numpy as jnp
from jax import lax
from jax.experimental import pallas as pl
from jax.experimental.pallas import tpu as pltpu
from jax.experimental.pallas import tpu_sc as plsc

NXV, NYV, NZV = 216, 248, 12
PC = (-0.16, -39.68, -3.08, 68.96, 39.68, 0.76)
PADH, PADW = 384.0, 1248.0
IMGH, IMGW = 370.0, 1242.0
NPTS = NXV * NYV * NZV
C = 64
HF, WF = 96, 312
QH, QW = HF + 1, WF + 2
VOFF = QH * QW

P = 128
NW = 32
BPW = 158
NBLK = NW * BPW
NPAD = NBLK * P
TCB = 16
TCG = NBLK // TCB


def _proj_body(prm_ref, idx0_ref, idx1_ref, wgt_ref):
    b = pl.program_id(0)
    i0 = lax.broadcasted_iota(jnp.int32, (TCB, P), 0)
    i1 = lax.broadcasted_iota(jnp.int32, (TCB, P), 1)
    n = (b * TCB + i0) * P + i1
    zi = n % NZV
    r = n // NZV
    yi = r % NYV
    xi = r // NYV
    f32 = jnp.float32
    xf = f32(PC[0]) + ((xi.astype(f32) + 0.5) * f32(PC[3] - PC[0])) / f32(NXV)
    yf = f32(PC[1]) + ((yi.astype(f32) + 0.5) * f32(PC[4] - PC[1])) / f32(NYV)
    zf = f32(PC[2]) + ((zi.astype(f32) + 0.5) * f32(PC[5] - PC[2])) / f32(NZV)
    xf = xf.astype(jnp.bfloat16).astype(f32)
    yf = yf.astype(jnp.bfloat16).astype(f32)
    zf = zf.astype(jnp.bfloat16).astype(f32)

    valids = []
    widx = []
    wgts = []
    for v in range(2):
        m = [prm_ref[v, j] for j in range(12)]
        sx, sy, ox, oy = (prm_ref[v, 12], prm_ref[v, 13],
                          prm_ref[v, 14], prm_ref[v, 15])
        p0 = m[0] * xf + m[1] * yf + m[2] * zf + m[3]
        p1 = m[4] * xf + m[5] * yf + m[6] * zf + m[7]
        p2 = m[8] * xf + m[9] * yf + m[10] * zf + m[11]
        d = jnp.where(jnp.abs(p2) < 1e-6, f32(1e-6), p2)
        px = p0 / d * sx - ox
        py = p1 / d * sy - oy
        valid = ((p2 > 1e-6) & (px >= 0) & (px < f32(IMGW)) &
                 (py >= 0) & (py < f32(IMGH)))
        valids.append(valid.astype(f32))
        u = px / f32(PADW) * f32(WF) - 0.5
        vv = py / f32(PADH) * f32(HF) - 0.5
        x0 = jnp.floor(u)
        y0 = jnp.floor(vv)
        xx = jnp.clip(x0 + 1.0, 0.0, f32(QW - 1)).astype(jnp.int32)
        yy = jnp.clip(y0 + 1.0, 0.0, f32(QH - 1)).astype(jnp.int32)
        widx.append(yy * QW + xx + v * VOFF)
        for dy, dx in ((0.0, 0.0), (0.0, 1.0), (1.0, 0.0), (1.0, 1.0)):
            xc = x0 + dx
            yc = y0 + dy
            w = (1.0 - jnp.abs(u - xc)) * (1.0 - jnp.abs(vv - yc))
            inb = ((xc >= 0) & (xc <= f32(WF - 1)) &
                   (yc >= 0) & (yc <= f32(HF - 1)))
            wgts.append(w * inb.astype(f32) * valids[v])

    cnt = valids[0] + valids[1]
    scale = 1.0 / jnp.clip(cnt, 0.001)
    idx0_ref[...] = widx[0]
    idx1_ref[...] = widx[1]
    for slot in range(8):
        wgt_ref[:, slot, :] = wgts[slot] * scale


def _run_proj(prm):
    out_shape = (jax.ShapeDtypeStruct((NBLK, P), jnp.int32),
                 jax.ShapeDtypeStruct((NBLK, P), jnp.int32),
                 jax.ShapeDtypeStruct((NBLK, 8, P), jnp.float32))
    return pl.pallas_call(
        _proj_body,
        grid=(TCG,),
        in_specs=[pl.BlockSpec(memory_space=pltpu.SMEM)],
        out_specs=(pl.BlockSpec((TCB, P), lambda i: (i, 0)),
                   pl.BlockSpec((TCB, P), lambda i: (i, 0)),
                   pl.BlockSpec((TCB, 8, P), lambda i: (i, 0, 0))),
        out_shape=out_shape,
    )(prm)


def _sc_body(idx0_hbm, idx1_hbm, wgt_hbm, tab_hbm, out_hbm,
             idxv, wgtv, rowsv, outv, sem_iw0, sem_iw1, sem_g0, sem_g1,
             sem_o0, sem_o1):
    sem_iw = (sem_iw0, sem_iw1)
    sem_g = (sem_g0, sem_g1)
    sem_o = (sem_o0, sem_o1)
    cid = lax.axis_index("c")
    sid = lax.axis_index("s")
    wid = sid * 2 + cid
    blk0 = wid * BPW

    def fire_iw(s, blk):
        pltpu.async_copy(idx0_hbm.at[blk], idxv.at[s, pl.ds(0, P)], sem_iw[s])
        pltpu.async_copy(idx1_hbm.at[blk], idxv.at[s, pl.ds(P, P)], sem_iw[s])
        pltpu.async_copy(wgt_hbm.at[blk], wgtv.at[s], sem_iw[s])

    def wait_iw(s):
        pltpu.make_async_copy(idx0_hbm.at[0], idxv.at[s, pl.ds(0, P)],
                              sem_iw[s]).wait()
        pltpu.make_async_copy(idx1_hbm.at[0], idxv.at[s, pl.ds(P, P)],
                              sem_iw[s]).wait()
        pltpu.make_async_copy(wgt_hbm.at[0], wgtv.at[s], sem_iw[s]).wait()

    def fire_gather(s):
        pltpu.async_copy(tab_hbm.at[idxv.at[s]], rowsv.at[s], sem_g[s])

    def wait_gather(s):
        pltpu.make_async_copy(tab_hbm.at[idxv.at[s]], rowsv.at[s],
                              sem_g[s]).wait()

    def fire_out(s, blk):
        pltpu.async_copy(outv.at[s], out_hbm.at[pl.ds(blk * (P // 2), P // 2)],
                         sem_o[s])

    def wait_out(s):
        pltpu.make_async_copy(outv.at[s], out_hbm.at[pl.ds(0, P // 2)],
                              sem_o[s]).wait()

    def compute(s):
        himask = jnp.full((16,), -65536, dtype=jnp.int32)

        @pl.loop(0, P // 16)
        def _(g):
            base = g * 16
            wv = [wgtv[s, k, pl.ds(base, 16)] for k in range(8)]
            for i in range(16):
                p = base + i
                accs = [None] * 4
                for k in range(8):
                    row = p if k < 4 else P + p
                    kk = k % 4
                    w0 = rowsv[s, row, pl.ds(kk * 32, 16)]
                    w1 = rowsv[s, row, pl.ds(kk * 32 + 16, 16)]
                    vals = (plsc.bitcast(w0 << 16, jnp.float32),
                            plsc.bitcast(w1 << 16, jnp.float32),
                            plsc.bitcast(w0 & himask, jnp.float32),
                            plsc.bitcast(w1 & himask, jnp.float32))
                    for jj in range(4):
                        t = wv[k][i] * vals[jj]
                        accs[jj] = t if accs[jj] is None else accs[jj] + t
                for jj in range(4):
                    outv[s, g * 8 + i // 2,
                         pl.ds((i % 2) * C + jj * 16, 16)] = accs[jj]

    def process(s, j):
        bj = j + s
        blk = blk0 + bj

        if s == 0:
            wait_iw(1)
            fire_gather(1)
        else:
            @pl.when(j < BPW - 2)
            def _():
                wait_iw(0)
                fire_gather(0)

        wait_gather(s)

        @pl.when(bj >= 2)
        def _():
            wait_out(s)

        compute(s)
        fire_out(s, blk)

        @pl.when(j < BPW - 2)
        def _():
            fire_iw(s, blk + 2)

    fire_iw(0, blk0)
    fire_iw(1, blk0 + 1)
    wait_iw(0)
    fire_gather(0)

    @pl.loop(0, BPW, step=2)
    def _(j):
        process(0, j)
        process(1, j)

    wait_out(0)
    wait_out(1)


@functools.cache
def _sc_gather_fn():
    mesh = plsc.VectorSubcoreMesh(core_axis_name="c", subcore_axis_name="s",
                                  num_cores=2, num_subcores=16)
    return functools.partial(
        pl.kernel,
        out_type=jax.ShapeDtypeStruct((NPAD // 2, 2 * C), jnp.float32),
        mesh=mesh,
        compiler_params=pltpu.CompilerParams(use_tc_tiling_on_sc=False,
                                             needs_layout_passes=False),
        scratch_types=[
            pltpu.VMEM((2, 2 * P), jnp.int32),
            pltpu.VMEM((2, 8, P), jnp.float32),
            pltpu.VMEM((2, 2 * P, 2 * C), jnp.int32),
            pltpu.VMEM((2, P // 2, 2 * C), jnp.float32),
            pltpu.SemaphoreType.DMA,
            pltpu.SemaphoreType.DMA,
            pltpu.SemaphoreType.DMA,
            pltpu.SemaphoreType.DMA,
            pltpu.SemaphoreType.DMA,
            pltpu.SemaphoreType.DMA,
        ],
    )(_sc_body)


def _build_table(feat):
    V = feat.shape[0]
    fp = jnp.pad(feat, ((0, 0), (0, 0), (1, 1), (1, 1)))
    q = jnp.concatenate([fp[:, :, 0:QH, 0:QW - 1], fp[:, :, 0:QH, 1:QW],
                         fp[:, :, 1:QH + 1, 0:QW - 1], fp[:, :, 1:QH + 1, 1:QW]],
                        axis=1)
    q = q.transpose(0, 2, 3, 1)
    q = q.reshape(V, QH, QW - 1, 4, 2, C // 2).swapaxes(-1, -2)
    q = q.astype(jnp.bfloat16)
    q = lax.bitcast_convert_type(q, jnp.int32)
    q = q.reshape(V, QH, QW - 1, 2 * C)
    q = jnp.pad(q, ((0, 0), (0, 0), (0, 1), (0, 0)))
    return q.reshape(V * QH * QW, 2 * C)


def kernel(x_fov, lidar2img, img_scale_factor, img_crop_offset):
    B, V, Cc, Hf, Wf = x_fov.shape
    vols = []
    for b in range(B):
        M = lax.optimization_barrier(
            lidar2img[b].astype(jnp.bfloat16)).astype(jnp.float32)
        prm = jnp.concatenate(
            [M[:, 0, :], M[:, 1, :], M[:, 2, :],
             jnp.broadcast_to(img_scale_factor[b][None, :], (V, 2)),
             jnp.broadcast_to(img_crop_offset[b][None, :], (V, 2))],
            axis=1)
        idx0, idx1, wgt = _run_proj(prm)
        tab = _build_table(x_fov[b])
        rows = _sc_gather_fn()(idx0, idx1, wgt, tab).reshape(NPAD, C)
        vol = rows[:NPTS].T.reshape(Cc, NXV, NYV, NZV)
        vols.append(vol)
    return jnp.stack(vols)


# speedup vs baseline: 1.4272x; 1.4272x over previous
import functools

import jax
import jax.numpy as jnp
from jax import lax
from jax.experimental import pallas as pl
from jax.experimental.pallas import tpu as pltpu
from jax.experimental.pallas import tpu_sc as plsc

NXV, NYV, NZV = 216, 248, 12
PC = (-0.16, -39.68, -3.08, 68.96, 39.68, 0.76)
PADH, PADW = 384.0, 1248.0
IMGH, IMGW = 370.0, 1242.0
NPTS = NXV * NYV * NZV
C = 64
HF, WF = 96, 312
QH, QW = HF + 1, WF + 2
VOFF = QH * QW

P = 128
NW = 32
BPW = 158
NBLK = NW * BPW
NPAD = NBLK * P
TCB = 16
TCG = NBLK // TCB


def _proj_body(prm_ref, idx0_ref, idx1_ref, wgt_ref):
    b = pl.program_id(0)
    i0 = lax.broadcasted_iota(jnp.int32, (TCB, P), 0)
    i1 = lax.broadcasted_iota(jnp.int32, (TCB, P), 1)
    n = (b * TCB + i0) * P + i1
    xi = n % NXV
    r = n // NXV
    yi = r % NYV
    zi = r // NYV
    f32 = jnp.float32
    xf = f32(PC[0]) + ((xi.astype(f32) + 0.5) * f32(PC[3] - PC[0])) / f32(NXV)
    yf = f32(PC[1]) + ((yi.astype(f32) + 0.5) * f32(PC[4] - PC[1])) / f32(NYV)
    zf = f32(PC[2]) + ((zi.astype(f32) + 0.5) * f32(PC[5] - PC[2])) / f32(NZV)
    xf = xf.astype(jnp.bfloat16).astype(f32)
    yf = yf.astype(jnp.bfloat16).astype(f32)
    zf = zf.astype(jnp.bfloat16).astype(f32)

    valids = []
    widx = []
    wgts = []
    for v in range(2):
        m = [prm_ref[v, j] for j in range(12)]
        sx, sy, ox, oy = (prm_ref[v, 12], prm_ref[v, 13],
                          prm_ref[v, 14], prm_ref[v, 15])
        p0 = m[0] * xf + m[1] * yf + m[2] * zf + m[3]
        p1 = m[4] * xf + m[5] * yf + m[6] * zf + m[7]
        p2 = m[8] * xf + m[9] * yf + m[10] * zf + m[11]
        d = jnp.where(jnp.abs(p2) < 1e-6, f32(1e-6), p2)
        px = p0 / d * sx - ox
        py = p1 / d * sy - oy
        valid = ((p2 > 1e-6) & (px >= 0) & (px < f32(IMGW)) &
                 (py >= 0) & (py < f32(IMGH)))
        valids.append(valid.astype(f32))
        u = px / f32(PADW) * f32(WF) - 0.5
        vv = py / f32(PADH) * f32(HF) - 0.5
        x0 = jnp.floor(u)
        y0 = jnp.floor(vv)
        xx = jnp.clip(x0 + 1.0, 0.0, f32(QW - 1)).astype(jnp.int32)
        yy = jnp.clip(y0 + 1.0, 0.0, f32(QH - 1)).astype(jnp.int32)
        widx.append(yy * QW + xx + v * VOFF)
        for dy, dx in ((0.0, 0.0), (0.0, 1.0), (1.0, 0.0), (1.0, 1.0)):
            xc = x0 + dx
            yc = y0 + dy
            w = (1.0 - jnp.abs(u - xc)) * (1.0 - jnp.abs(vv - yc))
            inb = ((xc >= 0) & (xc <= f32(WF - 1)) &
                   (yc >= 0) & (yc <= f32(HF - 1)))
            wgts.append(w * inb.astype(f32) * valids[v])

    cnt = valids[0] + valids[1]
    scale = 1.0 / jnp.clip(cnt, 0.001)
    idx0_ref[...] = widx[0]
    idx1_ref[...] = widx[1]
    for slot in range(8):
        wgt_ref[:, slot, :] = wgts[slot] * scale


def _run_proj(prm):
    out_shape = (jax.ShapeDtypeStruct((NBLK, P), jnp.int32),
                 jax.ShapeDtypeStruct((NBLK, P), jnp.int32),
                 jax.ShapeDtypeStruct((NBLK, 8, P), jnp.float32))
    return pl.pallas_call(
        _proj_body,
        grid=(TCG,),
        in_specs=[pl.BlockSpec(memory_space=pltpu.SMEM)],
        out_specs=(pl.BlockSpec((TCB, P), lambda i: (i, 0)),
                   pl.BlockSpec((TCB, P), lambda i: (i, 0)),
                   pl.BlockSpec((TCB, 8, P), lambda i: (i, 0, 0))),
        out_shape=out_shape,
    )(prm)


def _sc_body(idx0_hbm, idx1_hbm, wgt_hbm, tab_hbm, out_hbm,
             idxv, wgtv, rowsv, outv, sem_iw0, sem_iw1, sem_g0, sem_g1,
             sem_o0, sem_o1):
    sem_iw = (sem_iw0, sem_iw1)
    sem_g = (sem_g0, sem_g1)
    sem_o = (sem_o0, sem_o1)
    cid = lax.axis_index("c")
    sid = lax.axis_index("s")
    wid = sid * 2 + cid
    blk0 = wid * BPW

    def fire_iw(s, blk):
        pltpu.async_copy(idx0_hbm.at[blk], idxv.at[s, pl.ds(0, P)], sem_iw[s])
        pltpu.async_copy(idx1_hbm.at[blk], idxv.at[s, pl.ds(P, P)], sem_iw[s])
        pltpu.async_copy(wgt_hbm.at[blk], wgtv.at[s], sem_iw[s])

    def wait_iw(s):
        pltpu.make_async_copy(idx0_hbm.at[0], idxv.at[s, pl.ds(0, P)],
                              sem_iw[s]).wait()
        pltpu.make_async_copy(idx1_hbm.at[0], idxv.at[s, pl.ds(P, P)],
                              sem_iw[s]).wait()
        pltpu.make_async_copy(wgt_hbm.at[0], wgtv.at[s], sem_iw[s]).wait()

    def fire_gather(s):
        pltpu.async_copy(tab_hbm.at[idxv.at[s]], rowsv.at[s], sem_g[s])

    def wait_gather(s):
        pltpu.make_async_copy(tab_hbm.at[idxv.at[s]], rowsv.at[s],
                              sem_g[s]).wait()

    def fire_out(s, blk):
        pltpu.async_copy(outv.at[s], out_hbm.at[pl.ds(blk * (P // 2), P // 2)],
                         sem_o[s])

    def wait_out(s):
        pltpu.make_async_copy(outv.at[s], out_hbm.at[pl.ds(0, P // 2)],
                              sem_o[s]).wait()

    def compute(s):
        himask = jnp.full((16,), -65536, dtype=jnp.int32)

        @pl.loop(0, P // 16)
        def _(g):
            base = g * 16
            wv = [wgtv[s, k, pl.ds(base, 16)] for k in range(8)]
            for i in range(16):
                p = base + i
                accs = [None] * 4
                for k in range(8):
                    row = p if k < 4 else P + p
                    kk = k % 4
                    w0 = rowsv[s, row, pl.ds(kk * 32, 16)]
                    w1 = rowsv[s, row, pl.ds(kk * 32 + 16, 16)]
                    vals = (plsc.bitcast(w0 << 16, jnp.float32),
                            plsc.bitcast(w1 << 16, jnp.float32),
                            plsc.bitcast(w0 & himask, jnp.float32),
                            plsc.bitcast(w1 & himask, jnp.float32))
                    for jj in range(4):
                        t = wv[k][i] * vals[jj]
                        accs[jj] = t if accs[jj] is None else accs[jj] + t
                for jj in range(4):
                    outv[s, g * 8 + i // 2,
                         pl.ds((i % 2) * C + jj * 16, 16)] = accs[jj]

    def process(s, j):
        bj = j + s
        blk = blk0 + bj

        if s == 0:
            wait_iw(1)
            fire_gather(1)
        else:
            @pl.when(j < BPW - 2)
            def _():
                wait_iw(0)
                fire_gather(0)

        wait_gather(s)

        @pl.when(bj >= 2)
        def _():
            wait_out(s)

        compute(s)
        fire_out(s, blk)

        @pl.when(j < BPW - 2)
        def _():
            fire_iw(s, blk + 2)

    fire_iw(0, blk0)
    fire_iw(1, blk0 + 1)
    wait_iw(0)
    fire_gather(0)

    @pl.loop(0, BPW, step=2)
    def _(j):
        process(0, j)
        process(1, j)

    wait_out(0)
    wait_out(1)


@functools.cache
def _sc_gather_fn():
    mesh = plsc.VectorSubcoreMesh(core_axis_name="c", subcore_axis_name="s",
                                  num_cores=2, num_subcores=16)
    return functools.partial(
        pl.kernel,
        out_type=jax.ShapeDtypeStruct((NPAD // 2, 2 * C), jnp.float32),
        mesh=mesh,
        compiler_params=pltpu.CompilerParams(use_tc_tiling_on_sc=False,
                                             needs_layout_passes=False),
        scratch_types=[
            pltpu.VMEM((2, 2 * P), jnp.int32),
            pltpu.VMEM((2, 8, P), jnp.float32),
            pltpu.VMEM((2, 2 * P, 2 * C), jnp.int32),
            pltpu.VMEM((2, P // 2, 2 * C), jnp.float32),
            pltpu.SemaphoreType.DMA,
            pltpu.SemaphoreType.DMA,
            pltpu.SemaphoreType.DMA,
            pltpu.SemaphoreType.DMA,
            pltpu.SemaphoreType.DMA,
            pltpu.SemaphoreType.DMA,
        ],
    )(_sc_body)


def _build_table(feat):
    V = feat.shape[0]
    fp = jnp.pad(feat, ((0, 0), (0, 0), (1, 1), (1, 1)))
    q = jnp.concatenate([fp[:, :, 0:QH, 0:QW - 1], fp[:, :, 0:QH, 1:QW],
                         fp[:, :, 1:QH + 1, 0:QW - 1], fp[:, :, 1:QH + 1, 1:QW]],
                        axis=1)
    q = q.transpose(0, 2, 3, 1)
    q = q.reshape(V, QH, QW - 1, 4, 2, C // 2).swapaxes(-1, -2)
    q = q.astype(jnp.bfloat16)
    q = lax.bitcast_convert_type(q, jnp.int32)
    q = q.reshape(V, QH, QW - 1, 2 * C)
    q = jnp.pad(q, ((0, 0), (0, 0), (0, 1), (0, 0)))
    return q.reshape(V * QH * QW, 2 * C)


def kernel(x_fov, lidar2img, img_scale_factor, img_crop_offset):
    B, V, Cc, Hf, Wf = x_fov.shape
    vols = []
    for b in range(B):
        M = lax.optimization_barrier(
            lidar2img[b].astype(jnp.bfloat16)).astype(jnp.float32)
        prm = jnp.concatenate(
            [M[:, 0, :], M[:, 1, :], M[:, 2, :],
             jnp.broadcast_to(img_scale_factor[b][None, :], (V, 2)),
             jnp.broadcast_to(img_crop_offset[b][None, :], (V, 2))],
            axis=1)
        idx0, idx1, wgt = _run_proj(prm)
        tab = _build_table(x_fov[b])
        rows = _sc_gather_fn()(idx0, idx1, wgt, tab).reshape(NPAD, C)
        vol = rows[:NPTS].reshape(NZV, NYV, NXV, Cc).transpose(3, 2, 1, 0)
        vols.append(vol)
    return jnp.stack(vols)
